# baseline (speedup 1.0000x reference)
import functools

import jax
import jax.numpy as jnp
from jax import lax
from jax.experimental import pallas as pl
from jax.experimental.pallas import tpu as pltpu
from jax.experimental.pallas import tpu_sc as plsc

N = 10000
D = 256
DH = 128
DHP = 144
E = 160000
EPAD = 163840
CHUNK = EPAD // 16
PCH = 64
SUP = 512
SCH = SUP // PCH
NSUPC = CHUNK // SUP + 1
LCAP = NSUPC * SUP
NACC = 10000
SLICE = NACC // 16
RB = 1000
GRID = N // RB
MASKBIT = 1 << 30

_f32 = jnp.float32


def _leaky(v):
    return jnp.where(v >= 0.0, v, 0.2 * v)


def _dense1_body(x_ref, w_ref, asw_ref, adw_ref,
                 h0_ref, h1_ref, as_ref, ad_ref, sum_s, sq_s):
    p = pl.program_id(0)
    i = pl.program_id(1)

    @pl.when(p == 0)
    def _():
        xb = x_ref[...]
        s = jnp.sum(xb, axis=0, keepdims=True)
        q = jnp.sum(xb * xb, axis=0, keepdims=True)

        @pl.when(i == 0)
        def _():
            sum_s[...] = s
            sq_s[...] = q

        @pl.when(i != 0)
        def _():
            sum_s[...] = sum_s[...] + s
            sq_s[...] = sq_s[...] + q

    @pl.when(p == 1)
    def _():
        mu = sum_s[...] * (1.0 / N)
        var = sq_s[...] * (1.0 / N) - mu * mu
        inv = lax.rsqrt(var + 1e-5)
        xn = (x_ref[...] - mu) * inv
        h = jnp.dot(xn, w_ref[...], preferred_element_type=_f32)
        ones = jnp.ones((h.shape[0], DHP - DH), _f32)
        h0_ref[...] = jnp.concatenate([h[:, :DH], ones], axis=1)
        h1_ref[...] = jnp.concatenate([h[:, DH:], ones], axis=1)
        as_ref[...] = jnp.dot(h, asw_ref[...], preferred_element_type=_f32)
        ad_ref[...] = jnp.dot(h, adw_ref[...], preferred_element_type=_f32)


def _dense2_body(c0_ref, c1_ref, as1_ref, ad1_ref, h0_ref, h1_ref,
                 bias_ref, w_ref, asw_ref, adw_ref,
                 g0_ref, g1_ref, as2_ref, ad2_ref):
    wself = jnp.exp(_leaky(as1_ref[...] + ad1_ref[...]))
    den = c0_ref[:, DH:DH + 1] + wself + 1e-16
    hb = jnp.concatenate([h0_ref[:, :DH], h1_ref[:, :DH]], axis=1)
    cb = jnp.concatenate([c0_ref[:, :DH], c1_ref[:, :DH]], axis=1)
    out1 = (cb + wself * hb) / den + bias_ref[0:1, :]
    h2 = jnp.dot(out1, w_ref[...], preferred_element_type=_f32)
    ones = jnp.ones((h2.shape[0], DHP - DH), _f32)
    g0_ref[...] = jnp.concatenate([h2[:, :DH], ones], axis=1)
    g1_ref[...] = jnp.concatenate([h2[:, DH:], ones], axis=1)
    as2_ref[...] = jnp.dot(h2, asw_ref[...], preferred_element_type=_f32)
    ad2_ref[...] = jnp.dot(h2, adw_ref[...], preferred_element_type=_f32)


def _final_body(c0_ref, c1_ref, as2_ref, ad2_ref, h0_ref, h1_ref,
                bias_ref, out_ref):
    wself = jnp.exp(_leaky(as2_ref[...] + ad2_ref[...]))
    den = c0_ref[:, DH:DH + 1] + wself + 1e-16
    hb = jnp.concatenate([h0_ref[:, :DH], h1_ref[:, :DH]], axis=1)
    cb = jnp.concatenate([c0_ref[:, :DH], c1_ref[:, :DH]], axis=1)
    out_ref[...] = (cb + wself * hb) / den + bias_ref[0:1, :]


def _row_spec(width=D):
    return pl.BlockSpec((RB, width), lambda i: (i, 0))


def _full_spec(shape):
    return pl.BlockSpec(shape, lambda i: tuple(0 for _ in shape))


def _row_spec2(width):
    return pl.BlockSpec((RB, width), lambda p, i: (i, 0))


_dense1_call = pl.pallas_call(
    _dense1_body,
    grid=(2, GRID),
    in_specs=[_row_spec2(D),
              pl.BlockSpec((D, D), lambda p, i: (0, 0)),
              pl.BlockSpec((D, 1), lambda p, i: (0, 0)),
              pl.BlockSpec((D, 1), lambda p, i: (0, 0))],
    out_specs=[_row_spec2(DHP), _row_spec2(DHP), _row_spec2(1),
               _row_spec2(1)],
    out_shape=[jax.ShapeDtypeStruct((N, DHP), _f32),
               jax.ShapeDtypeStruct((N, DHP), _f32),
               jax.ShapeDtypeStruct((N, 1), _f32),
               jax.ShapeDtypeStruct((N, 1), _f32)],
    scratch_shapes=[pltpu.VMEM((1, D), _f32), pltpu.VMEM((1, D), _f32)],
)

_dense2_call = pl.pallas_call(
    _dense2_body,
    grid=(GRID,),
    in_specs=[_row_spec(DHP), _row_spec(DHP), _row_spec(1), _row_spec(1),
              _row_spec(DHP), _row_spec(DHP),
              _full_spec((1, D)), _full_spec((D, D)),
              _full_spec((D, 1)), _full_spec((D, 1))],
    out_specs=[_row_spec(DHP), _row_spec(DHP), _row_spec(1), _row_spec(1)],
    out_shape=[jax.ShapeDtypeStruct((N, DHP), _f32),
               jax.ShapeDtypeStruct((N, DHP), _f32),
               jax.ShapeDtypeStruct((N, 1), _f32),
               jax.ShapeDtypeStruct((N, 1), _f32)],
)

_final_call = pl.pallas_call(
    _final_body,
    grid=(GRID,),
    in_specs=[_row_spec(DHP), _row_spec(DHP), _row_spec(1), _row_spec(1),
              _row_spec(DHP), _row_spec(DHP), _full_spec((1, D))],
    out_specs=_row_spec(),
    out_shape=jax.ShapeDtypeStruct((N, D), _f32),
)


def _part_body(src_hbm, dst_hbm, addr_hbm,
               src0_hbm, dst0_hbm, cnt0_hbm, src1_hbm, dst1_hbm, cnt1_hbm,
               src_v, dst_v, addr_v, src0_v, dst0_v, src1_v, dst1_v, cnt_v):
    c = lax.axis_index("c")
    t = lax.axis_index("s")

    @pl.when(c == 0)
    def _():
        pltpu.sync_copy(src_hbm.at[t], src_v)
        pltpu.sync_copy(dst_hbm.at[t], dst_v)
        pltpu.sync_copy(addr_hbm.at[t], addr_v)

        def cstep(i, pos):
            pos0, pos1 = pos
            off = i * 16
            sv = src_v[pl.ds(off, 16)]
            dv = dst_v[pl.ds(off, 16)]
            av = addr_v[pl.ds(off, 16)]
            m0 = av == 0
            m1 = av == 1
            plsc.store_compressed(src0_v.at[pl.ds(pos0, 16)], sv, mask=m0)
            plsc.store_compressed(dst0_v.at[pl.ds(pos0, 16)], dv, mask=m0)
            plsc.store_compressed(src1_v.at[pl.ds(pos1, 16)], sv, mask=m1)
            plsc.store_compressed(dst1_v.at[pl.ds(pos1, 16)], dv, mask=m1)
            n0 = plsc.all_reduce_population_count(m0)[0]
            n1 = plsc.all_reduce_population_count(m1)[0]
            return pos0 + n0, pos1 + n1

        pos0, pos1 = lax.fori_loop(0, CHUNK // 16, cstep, (0, 0))

        sent = jnp.full((16,), MASKBIT, jnp.int32)
        zero = jnp.zeros((16,), jnp.int32)
        for k in range(SUP // 16):
            src0_v[pl.ds(pos0 + k * 16, 16)] = sent
            dst0_v[pl.ds(pos0 + k * 16, 16)] = zero
            src1_v[pl.ds(pos1 + k * 16, 16)] = sent
            dst1_v[pl.ds(pos1 + k * 16, 16)] = zero

        pltpu.sync_copy(src0_v, src0_hbm.at[t])
        pltpu.sync_copy(dst0_v, dst0_hbm.at[t])
        pltpu.sync_copy(src1_v, src1_hbm.at[t])
        pltpu.sync_copy(dst1_v, dst1_hbm.at[t])
        cnt_v[...] = jnp.full((16,), (pos0 + SUP - 1) // SUP, jnp.int32)
        pltpu.sync_copy(cnt_v, cnt0_hbm.at[t])
        cnt_v[...] = jnp.full((16,), (pos1 + SUP - 1) // SUP, jnp.int32)
        pltpu.sync_copy(cnt_v, cnt1_hbm.at[t])


_part_call = pl.kernel(
    _part_body,
    out_type=[jax.ShapeDtypeStruct((16, LCAP), jnp.int32),
              jax.ShapeDtypeStruct((16, LCAP), jnp.int32),
              jax.ShapeDtypeStruct((16, 16), jnp.int32),
              jax.ShapeDtypeStruct((16, LCAP), jnp.int32),
              jax.ShapeDtypeStruct((16, LCAP), jnp.int32),
              jax.ShapeDtypeStruct((16, 16), jnp.int32)],
    mesh=plsc.VectorSubcoreMesh(core_axis_name="c", subcore_axis_name="s"),
    compiler_params=pltpu.CompilerParams(needs_layout_passes=False,
                                         use_tc_tiling_on_sc=False),
    scratch_types=[
        pltpu.VMEM((CHUNK,), jnp.int32),
        pltpu.VMEM((CHUNK,), jnp.int32),
        pltpu.VMEM((CHUNK,), jnp.int32),
        pltpu.VMEM((LCAP,), jnp.int32),
        pltpu.VMEM((LCAP,), jnp.int32),
        pltpu.VMEM((LCAP,), jnp.int32),
        pltpu.VMEM((LCAP,), jnp.int32),
        pltpu.VMEM((16,), jnp.int32),
    ],
)


def _sc_body(psd_hbm, cnt_hbm, asrc_hbm, adst_hbm, h0_hbm, h1_hbm,
             c0_hbm, c1_hbm,
             psd_s, w_s, asrc_v, adst_v, rows_a, rows_b, cnt_v,
             sem_ga, sem_gb, sem_sa, sem_sb, acc_sh):
    c = lax.axis_index("c")
    t = lax.axis_index("s")

    pltpu.sync_copy(asrc_hbm, asrc_v)
    pltpu.sync_copy(adst_hbm, adst_v)
    pltpu.sync_copy(cnt_hbm.at[t], cnt_v)
    nsup = cnt_v[...][0]

    def z2(r, _):
        for k in range(DHP // 16):
            rows_a[r, pl.ds(k * 16, 16)] = jnp.zeros((16,), _f32)
        return 0

    lax.fori_loop(0, PCH, z2, 0)
    part = 0
    while part < SLICE:
        sz = min(PCH, SLICE - part)
        pltpu.sync_copy(rows_a.at[pl.ds(0, sz)],
                        acc_sh.at[pl.ds(t * SLICE + part, sz)])
        part += sz
    plsc.subcore_barrier()

    rows = (rows_a, rows_b)
    gsem = (sem_ga, sem_gb)
    ssem = (sem_sa, sem_sb)

    def _gather(j, buf, sem):
        idx = psd_s.at[0, j]
        d0 = pltpu.make_async_copy(h0_hbm.at[idx], buf, sem)
        d1 = pltpu.make_async_copy(h1_hbm.at[idx], buf, sem)

        @pl.when(c == 0)
        def _():
            d0.start()

        @pl.when(c == 1)
        def _():
            d1.start()

        return d0

    def sup_step(sup, _):
        pltpu.sync_copy(psd_hbm.at[t, sup], psd_s)

        def wstep(i, _):
            r = i // 4
            o = pl.ds((i % 4) * 16, 16)
            pv = psd_s[0, r, o]
            srcv = jnp.bitwise_and(pv, MASKBIT - 1)
            maskv = pv < MASKBIT
            av = plsc.load_gather(asrc_v, [srcv])
            bv = plsc.load_gather(adst_v, [psd_s[1, r, o]])
            e = av + bv
            e = jnp.where(e >= 0.0, e, 0.2 * e)
            w_s[pl.ds(i * 16, 16)] = jnp.where(maskv, jnp.exp(e), 0.0)
            psd_s[0, r, o] = srcv
            return 0

        lax.fori_loop(0, SUP // 16, wstep, 0)

        gw = [None, None]
        sw = [None, None]
        gw[0] = _gather(0, rows[0], gsem[0])
        for j in range(SCH):
            b = j & 1
            gw[b].wait()
            if j + 1 < SCH:
                if sw[1 - b] is not None:
                    sw[1 - b].wait()
                gw[1 - b] = _gather(j + 1, rows[1 - b], gsem[1 - b])

            def sstep(i, _, _b=b, _j=j):
                wv = w_s[pl.ds(_j * PCH + i * 16, 16)]
                for jj in range(16):
                    wj = wv[jj]
                    r = i * 16 + jj
                    for k in range(DHP // 16):
                        o = pl.ds(k * 16, 16)
                        rows[_b][r, o] = rows[_b][r, o] * wj
                return 0

            lax.fori_loop(0, PCH // 16, sstep, 0)
            sw[b] = pltpu.async_copy(rows[b], acc_sh.at[psd_s.at[1, j]],
                                     ssem[b], add=True)
        sw[0].wait()
        sw[1].wait()
        return 0

    lax.fori_loop(0, nsup, sup_step, 0)
    plsc.subcore_barrier()

    wsl = pl.ds(t * SLICE, SLICE)

    @pl.when(c == 0)
    def _():
        pltpu.sync_copy(acc_sh.at[wsl], c0_hbm.at[wsl])

    @pl.when(c == 1)
    def _():
        pltpu.sync_copy(acc_sh.at[wsl], c1_hbm.at[wsl])


_sc_call = pl.kernel(
    _sc_body,
    out_type=[jax.ShapeDtypeStruct((NACC, DHP), _f32),
              jax.ShapeDtypeStruct((NACC, DHP), _f32)],
    mesh=plsc.VectorSubcoreMesh(core_axis_name="c", subcore_axis_name="s"),
    compiler_params=pltpu.CompilerParams(needs_layout_passes=False,
                                         use_tc_tiling_on_sc=False),
    scratch_types=[
        pltpu.VMEM((2, SCH, PCH), jnp.int32),
        pltpu.VMEM((SUP,), _f32),
        pltpu.VMEM((N,), _f32),
        pltpu.VMEM((N,), _f32),
        pltpu.VMEM((PCH, DHP), _f32),
        pltpu.VMEM((PCH, DHP), _f32),
        pltpu.VMEM((16,), jnp.int32),
        pltpu.SemaphoreType.DMA,
        pltpu.SemaphoreType.DMA,
        pltpu.SemaphoreType.DMA,
        pltpu.SemaphoreType.DMA,
        pltpu.VMEM_SHARED((NACC, DHP), _f32),
    ],
)


def kernel(x, edge_index, edge_addr, W_inc, att_src_inc, att_dst_inc,
           bias_inc, W_near, att_src_near, att_dst_near, bias_near):
    src = edge_index[0]
    dst = edge_index[1]
    pad = EPAD - E
    srcp = jnp.concatenate([src, jnp.zeros((pad,), jnp.int32)])
    dstp = jnp.concatenate([dst, jnp.zeros((pad,), jnp.int32)])
    addrp = jnp.concatenate([edge_addr, jnp.full((pad,), 2, jnp.int32)])

    src0, dst0, cnt0, src1, dst1, cnt1 = _part_call(
        srcp.reshape(16, CHUNK), dstp.reshape(16, CHUNK),
        addrp.reshape(16, CHUNK))
    psd0 = jnp.stack([src0.reshape(16, NSUPC, SCH, PCH),
                      dst0.reshape(16, NSUPC, SCH, PCH)], axis=2)
    psd1 = jnp.stack([src1.reshape(16, NSUPC, SCH, PCH),
                      dst1.reshape(16, NSUPC, SCH, PCH)], axis=2)

    h0, h1, a_s1, a_d1 = _dense1_call(
        x, W_inc, att_src_inc.reshape(D, 1), att_dst_inc.reshape(D, 1))

    c0, c1 = _sc_call(psd0, cnt0, a_s1.reshape(N), a_d1.reshape(N), h0, h1)

    g0, g1, a_s2, a_d2 = _dense2_call(
        c0[:N], c1[:N], a_s1, a_d1, h0, h1,
        bias_inc.reshape(1, D), W_near, att_src_near.reshape(D, 1),
        att_dst_near.reshape(D, 1))

    e0, e1 = _sc_call(psd1, cnt1, a_s2.reshape(N), a_d2.reshape(N), g0, g1)

    out = _final_call(e0[:N], e1[:N], a_s2, a_d2,
                      g0, g1, bias_near.reshape(1, D))
    return out


# speedup vs baseline: 1.0452x; 1.0452x over previous
import functools

import jax
import jax.numpy as jnp
from jax import lax
from jax.experimental import pallas as pl
from jax.experimental.pallas import tpu as pltpu
from jax.experimental.pallas import tpu_sc as plsc

N = 10000
D = 256
DH = 128
DHP = 144
E = 160000
EPAD = 163840
CHUNK = EPAD // 16
PCH = 128
SUP = 512
SCH = SUP // PCH
NSUPC = CHUNK // SUP + 1
LCAP = NSUPC * SUP
NACC = 10000
SLICE = NACC // 16
RB = 1000
GRID = N // RB
MASKBIT = 1 << 30

_f32 = jnp.float32


def _leaky(v):
    return jnp.where(v >= 0.0, v, 0.2 * v)


def _dense1_body(x_ref, w_ref, asw_ref, adw_ref,
                 h0_ref, h1_ref, as_ref, ad_ref, sum_s, sq_s):
    p = pl.program_id(0)
    i = pl.program_id(1)

    @pl.when(p == 0)
    def _():
        xb = x_ref[...]
        s = jnp.sum(xb, axis=0, keepdims=True)
        q = jnp.sum(xb * xb, axis=0, keepdims=True)

        @pl.when(i == 0)
        def _():
            sum_s[...] = s
            sq_s[...] = q

        @pl.when(i != 0)
        def _():
            sum_s[...] = sum_s[...] + s
            sq_s[...] = sq_s[...] + q

    @pl.when(p == 1)
    def _():
        mu = sum_s[...] * (1.0 / N)
        var = sq_s[...] * (1.0 / N) - mu * mu
        inv = lax.rsqrt(var + 1e-5)
        xn = (x_ref[...] - mu) * inv
        h = jnp.dot(xn, w_ref[...], preferred_element_type=_f32)
        ones = jnp.ones((h.shape[0], DHP - DH), _f32)
        h0_ref[...] = jnp.concatenate([h[:, :DH], ones], axis=1)
        h1_ref[...] = jnp.concatenate([h[:, DH:], ones], axis=1)
        as_ref[...] = jnp.dot(h, asw_ref[...], preferred_element_type=_f32)
        ad_ref[...] = jnp.dot(h, adw_ref[...], preferred_element_type=_f32)


def _dense2_body(c0_ref, c1_ref, as1_ref, ad1_ref, h0_ref, h1_ref,
                 bias_ref, w_ref, asw_ref, adw_ref,
                 g0_ref, g1_ref, as2_ref, ad2_ref):
    wself = jnp.exp(_leaky(as1_ref[...] + ad1_ref[...]))
    den = c0_ref[:, DH:DH + 1] + wself + 1e-16
    hb = jnp.concatenate([h0_ref[:, :DH], h1_ref[:, :DH]], axis=1)
    cb = jnp.concatenate([c0_ref[:, :DH], c1_ref[:, :DH]], axis=1)
    out1 = (cb + wself * hb) / den + bias_ref[0:1, :]
    h2 = jnp.dot(out1, w_ref[...], preferred_element_type=_f32)
    ones = jnp.ones((h2.shape[0], DHP - DH), _f32)
    g0_ref[...] = jnp.concatenate([h2[:, :DH], ones], axis=1)
    g1_ref[...] = jnp.concatenate([h2[:, DH:], ones], axis=1)
    as2_ref[...] = jnp.dot(h2, asw_ref[...], preferred_element_type=_f32)
    ad2_ref[...] = jnp.dot(h2, adw_ref[...], preferred_element_type=_f32)


def _final_body(c0_ref, c1_ref, as2_ref, ad2_ref, h0_ref, h1_ref,
                bias_ref, out_ref):
    wself = jnp.exp(_leaky(as2_ref[...] + ad2_ref[...]))
    den = c0_ref[:, DH:DH + 1] + wself + 1e-16
    hb = jnp.concatenate([h0_ref[:, :DH], h1_ref[:, :DH]], axis=1)
    cb = jnp.concatenate([c0_ref[:, :DH], c1_ref[:, :DH]], axis=1)
    out_ref[...] = (cb + wself * hb) / den + bias_ref[0:1, :]


def _row_spec(width=D):
    return pl.BlockSpec((RB, width), lambda i: (i, 0))


def _full_spec(shape):
    return pl.BlockSpec(shape, lambda i: tuple(0 for _ in shape))


def _row_spec2(width):
    return pl.BlockSpec((RB, width), lambda p, i: (i, 0))


_dense1_call = pl.pallas_call(
    _dense1_body,
    grid=(2, GRID),
    in_specs=[_row_spec2(D),
              pl.BlockSpec((D, D), lambda p, i: (0, 0)),
              pl.BlockSpec((D, 1), lambda p, i: (0, 0)),
              pl.BlockSpec((D, 1), lambda p, i: (0, 0))],
    out_specs=[_row_spec2(DHP), _row_spec2(DHP), _row_spec2(1),
               _row_spec2(1)],
    out_shape=[jax.ShapeDtypeStruct((N, DHP), _f32),
               jax.ShapeDtypeStruct((N, DHP), _f32),
               jax.ShapeDtypeStruct((N, 1), _f32),
               jax.ShapeDtypeStruct((N, 1), _f32)],
    scratch_shapes=[pltpu.VMEM((1, D), _f32), pltpu.VMEM((1, D), _f32)],
)

_dense2_call = pl.pallas_call(
    _dense2_body,
    grid=(GRID,),
    in_specs=[_row_spec(DHP), _row_spec(DHP), _row_spec(1), _row_spec(1),
              _row_spec(DHP), _row_spec(DHP),
              _full_spec((1, D)), _full_spec((D, D)),
              _full_spec((D, 1)), _full_spec((D, 1))],
    out_specs=[_row_spec(DHP), _row_spec(DHP), _row_spec(1), _row_spec(1)],
    out_shape=[jax.ShapeDtypeStruct((N, DHP), _f32),
               jax.ShapeDtypeStruct((N, DHP), _f32),
               jax.ShapeDtypeStruct((N, 1), _f32),
               jax.ShapeDtypeStruct((N, 1), _f32)],
)

_final_call = pl.pallas_call(
    _final_body,
    grid=(GRID,),
    in_specs=[_row_spec(DHP), _row_spec(DHP), _row_spec(1), _row_spec(1),
              _row_spec(DHP), _row_spec(DHP), _full_spec((1, D))],
    out_specs=_row_spec(),
    out_shape=jax.ShapeDtypeStruct((N, D), _f32),
)


def _part_body(src_hbm, dst_hbm, addr_hbm,
               src0_hbm, dst0_hbm, cnt0_hbm, src1_hbm, dst1_hbm, cnt1_hbm,
               src_v, dst_v, addr_v, src0_v, dst0_v, src1_v, dst1_v, cnt_v):
    c = lax.axis_index("c")
    t = lax.axis_index("s")

    @pl.when(c == 0)
    def _():
        pltpu.sync_copy(src_hbm.at[t], src_v)
        pltpu.sync_copy(dst_hbm.at[t], dst_v)
        pltpu.sync_copy(addr_hbm.at[t], addr_v)

        def cstep(i, pos):
            pos0, pos1 = pos
            off = i * 16
            sv = src_v[pl.ds(off, 16)]
            dv = dst_v[pl.ds(off, 16)]
            av = addr_v[pl.ds(off, 16)]
            m0 = av == 0
            m1 = av == 1
            plsc.store_compressed(src0_v.at[pl.ds(pos0, 16)], sv, mask=m0)
            plsc.store_compressed(dst0_v.at[pl.ds(pos0, 16)], dv, mask=m0)
            plsc.store_compressed(src1_v.at[pl.ds(pos1, 16)], sv, mask=m1)
            plsc.store_compressed(dst1_v.at[pl.ds(pos1, 16)], dv, mask=m1)
            n0 = plsc.all_reduce_population_count(m0)[0]
            n1 = plsc.all_reduce_population_count(m1)[0]
            return pos0 + n0, pos1 + n1

        pos0, pos1 = lax.fori_loop(0, CHUNK // 16, cstep, (0, 0))

        sent = jnp.full((16,), MASKBIT, jnp.int32)
        zero = jnp.zeros((16,), jnp.int32)
        for k in range(SUP // 16):
            src0_v[pl.ds(pos0 + k * 16, 16)] = sent
            dst0_v[pl.ds(pos0 + k * 16, 16)] = zero
            src1_v[pl.ds(pos1 + k * 16, 16)] = sent
            dst1_v[pl.ds(pos1 + k * 16, 16)] = zero

        pltpu.sync_copy(src0_v, src0_hbm.at[t])
        pltpu.sync_copy(dst0_v, dst0_hbm.at[t])
        pltpu.sync_copy(src1_v, src1_hbm.at[t])
        pltpu.sync_copy(dst1_v, dst1_hbm.at[t])
        cnt_v[...] = jnp.full((16,), (pos0 + SUP - 1) // SUP, jnp.int32)
        pltpu.sync_copy(cnt_v, cnt0_hbm.at[t])
        cnt_v[...] = jnp.full((16,), (pos1 + SUP - 1) // SUP, jnp.int32)
        pltpu.sync_copy(cnt_v, cnt1_hbm.at[t])


_part_call = pl.kernel(
    _part_body,
    out_type=[jax.ShapeDtypeStruct((16, LCAP), jnp.int32),
              jax.ShapeDtypeStruct((16, LCAP), jnp.int32),
              jax.ShapeDtypeStruct((16, 16), jnp.int32),
              jax.ShapeDtypeStruct((16, LCAP), jnp.int32),
              jax.ShapeDtypeStruct((16, LCAP), jnp.int32),
              jax.ShapeDtypeStruct((16, 16), jnp.int32)],
    mesh=plsc.VectorSubcoreMesh(core_axis_name="c", subcore_axis_name="s"),
    compiler_params=pltpu.CompilerParams(needs_layout_passes=False,
                                         use_tc_tiling_on_sc=False),
    scratch_types=[
        pltpu.VMEM((CHUNK,), jnp.int32),
        pltpu.VMEM((CHUNK,), jnp.int32),
        pltpu.VMEM((CHUNK,), jnp.int32),
        pltpu.VMEM((LCAP,), jnp.int32),
        pltpu.VMEM((LCAP,), jnp.int32),
        pltpu.VMEM((LCAP,), jnp.int32),
        pltpu.VMEM((LCAP,), jnp.int32),
        pltpu.VMEM((16,), jnp.int32),
    ],
)


def _wc_body(psd_hbm, cnt_hbm, asrc_hbm, adst_hbm,
             wl_hbm, psdc_hbm,
             psd_s, w_s, asrc_v, adst_v, cnt_v):
    c = lax.axis_index("c")
    t = lax.axis_index("s")

    @pl.when(c == 0)
    def _():
        pltpu.sync_copy(asrc_hbm, asrc_v)
        pltpu.sync_copy(adst_hbm, adst_v)
        pltpu.sync_copy(cnt_hbm.at[t], cnt_v)
        nsup = cnt_v[...][0]

        def sup_step(sup, _):
            pltpu.sync_copy(psd_hbm.at[t, sup], psd_s)

            def wstep(i, _):
                r = i // (PCH // 16)
                o = pl.ds((i % (PCH // 16)) * 16, 16)
                pv = psd_s[0, r, o]
                srcv = jnp.bitwise_and(pv, MASKBIT - 1)
                maskv = pv < MASKBIT
                av = plsc.load_gather(asrc_v, [srcv])
                bv = plsc.load_gather(adst_v, [psd_s[1, r, o]])
                e = av + bv
                e = jnp.where(e >= 0.0, e, 0.2 * e)
                w_s[pl.ds(i * 16, 16)] = jnp.where(maskv, jnp.exp(e), 0.0)
                psd_s[0, r, o] = srcv
                return 0

            lax.fori_loop(0, SUP // 16, wstep, 0)
            pltpu.sync_copy(w_s, wl_hbm.at[t, sup])
            pltpu.sync_copy(psd_s, psdc_hbm.at[t, sup])
            return 0

        lax.fori_loop(0, nsup, sup_step, 0)


_wc_call = pl.kernel(
    _wc_body,
    out_type=[jax.ShapeDtypeStruct((16, NSUPC, SUP), _f32),
              jax.ShapeDtypeStruct((16, NSUPC, 2, SCH, PCH), jnp.int32)],
    mesh=plsc.VectorSubcoreMesh(core_axis_name="c", subcore_axis_name="s"),
    compiler_params=pltpu.CompilerParams(needs_layout_passes=False,
                                         use_tc_tiling_on_sc=False),
    scratch_types=[
        pltpu.VMEM((2, SCH, PCH), jnp.int32),
        pltpu.VMEM((SUP,), _f32),
        pltpu.VMEM((N,), _f32),
        pltpu.VMEM((N,), _f32),
        pltpu.VMEM((16,), jnp.int32),
    ],
)


def _sc_body(psd_hbm, wl_hbm, cnt_hbm, h0_hbm, h1_hbm,
             c0_hbm, c1_hbm,
             psd_s, w_s, rows_a, rows_b, cnt_v,
             sem_ga, sem_gb, sem_sa, sem_sb, acc_sh):
    c = lax.axis_index("c")
    t = lax.axis_index("s")

    pltpu.sync_copy(cnt_hbm.at[t], cnt_v)
    nsup = cnt_v[...][0]

    def z2(r, _):
        for k in range(DHP // 16):
            rows_a[r, pl.ds(k * 16, 16)] = jnp.zeros((16,), _f32)
        return 0

    lax.fori_loop(0, PCH, z2, 0)
    part = 0
    while part < SLICE:
        sz = min(PCH, SLICE - part)
        pltpu.sync_copy(rows_a.at[pl.ds(0, sz)],
                        acc_sh.at[pl.ds(t * SLICE + part, sz)])
        part += sz
    plsc.subcore_barrier()

    rows = (rows_a, rows_b)
    gsem = (sem_ga, sem_gb)
    ssem = (sem_sa, sem_sb)

    def _gather(j, buf, sem):
        idx = psd_s.at[0, j]
        d0 = pltpu.make_async_copy(h0_hbm.at[idx], buf, sem)
        d1 = pltpu.make_async_copy(h1_hbm.at[idx], buf, sem)

        @pl.when(c == 0)
        def _():
            d0.start()

        @pl.when(c == 1)
        def _():
            d1.start()

        return d0

    def sup_step(sup, _):
        pltpu.sync_copy(psd_hbm.at[t, sup], psd_s)
        pltpu.sync_copy(wl_hbm.at[t, sup], w_s)

        gw = [None, None]
        sw = [None, None]
        gw[0] = _gather(0, rows[0], gsem[0])
        for j in range(SCH):
            b = j & 1
            gw[b].wait()
            if j + 1 < SCH:
                if sw[1 - b] is not None:
                    sw[1 - b].wait()
                gw[1 - b] = _gather(j + 1, rows[1 - b], gsem[1 - b])

            def sstep(i, _, _b=b, _j=j):
                wv = w_s[pl.ds(_j * PCH + i * 16, 16)]
                for jj in range(16):
                    wj = wv[jj]
                    r = i * 16 + jj
                    for k in range(DHP // 16):
                        o = pl.ds(k * 16, 16)
                        rows[_b][r, o] = rows[_b][r, o] * wj
                return 0

            lax.fori_loop(0, PCH // 16, sstep, 0)
            sw[b] = pltpu.async_copy(rows[b], acc_sh.at[psd_s.at[1, j]],
                                     ssem[b], add=True)
        sw[0].wait()
        sw[1].wait()
        return 0

    lax.fori_loop(0, nsup, sup_step, 0)
    plsc.subcore_barrier()

    wsl = pl.ds(t * SLICE, SLICE)

    @pl.when(c == 0)
    def _():
        pltpu.sync_copy(acc_sh.at[wsl], c0_hbm.at[wsl])

    @pl.when(c == 1)
    def _():
        pltpu.sync_copy(acc_sh.at[wsl], c1_hbm.at[wsl])


_sc_call = pl.kernel(
    _sc_body,
    out_type=[jax.ShapeDtypeStruct((NACC, DHP), _f32),
              jax.ShapeDtypeStruct((NACC, DHP), _f32)],
    mesh=plsc.VectorSubcoreMesh(core_axis_name="c", subcore_axis_name="s"),
    compiler_params=pltpu.CompilerParams(needs_layout_passes=False,
                                         use_tc_tiling_on_sc=False),
    scratch_types=[
        pltpu.VMEM((2, SCH, PCH), jnp.int32),
        pltpu.VMEM((SUP,), _f32),
        pltpu.VMEM((PCH, DHP), _f32),
        pltpu.VMEM((PCH, DHP), _f32),
        pltpu.VMEM((16,), jnp.int32),
        pltpu.SemaphoreType.DMA,
        pltpu.SemaphoreType.DMA,
        pltpu.SemaphoreType.DMA,
        pltpu.SemaphoreType.DMA,
        pltpu.VMEM_SHARED((NACC, DHP), _f32),
    ],
)


def kernel(x, edge_index, edge_addr, W_inc, att_src_inc, att_dst_inc,
           bias_inc, W_near, att_src_near, att_dst_near, bias_near):
    src = edge_index[0]
    dst = edge_index[1]
    pad = EPAD - E
    srcp = jnp.concatenate([src, jnp.zeros((pad,), jnp.int32)])
    dstp = jnp.concatenate([dst, jnp.zeros((pad,), jnp.int32)])
    addrp = jnp.concatenate([edge_addr, jnp.full((pad,), 2, jnp.int32)])

    src0, dst0, cnt0, src1, dst1, cnt1 = _part_call(
        srcp.reshape(16, CHUNK), dstp.reshape(16, CHUNK),
        addrp.reshape(16, CHUNK))
    psd0 = jnp.stack([src0.reshape(16, NSUPC, SCH, PCH),
                      dst0.reshape(16, NSUPC, SCH, PCH)], axis=2)
    psd1 = jnp.stack([src1.reshape(16, NSUPC, SCH, PCH),
                      dst1.reshape(16, NSUPC, SCH, PCH)], axis=2)

    h0, h1, a_s1, a_d1 = _dense1_call(
        x, W_inc, att_src_inc.reshape(D, 1), att_dst_inc.reshape(D, 1))

    wl0, psdc0 = _wc_call(psd0, cnt0, a_s1.reshape(N), a_d1.reshape(N))
    c0, c1 = _sc_call(psdc0, wl0, cnt0, h0, h1)

    g0, g1, a_s2, a_d2 = _dense2_call(
        c0[:N], c1[:N], a_s1, a_d1, h0, h1,
        bias_inc.reshape(1, D), W_near, att_src_near.reshape(D, 1),
        att_dst_near.reshape(D, 1))

    wl1, psdc1 = _wc_call(psd1, cnt1, a_s2.reshape(N), a_d2.reshape(N))
    e0, e1 = _sc_call(psdc1, wl1, cnt1, g0, g1)

    out = _final_call(e0[:N], e1[:N], a_s2, a_d2,
                      g0, g1, bias_near.reshape(1, D))
    return out


# speedup vs baseline: 1.0456x; 1.0005x over previous
import jax
import jax.numpy as jnp
from jax import lax
from jax.experimental import pallas as pl
from jax.experimental.pallas import tpu as pltpu
from jax.experimental.pallas import tpu_sc as plsc

N = 10000
D = 256
DH = 128
DHP = 144
E = 160000
EPAD = 163840
CHUNK = EPAD // 16
PCH = 128
SUP = 512
SCH = SUP // PCH
NSUPC = CHUNK // SUP + 1
LCAP = NSUPC * SUP
NACC = 10000
SLICE = NACC // 16
RB = 1000
GRID = N // RB
MASKBIT = 1 << 30

_f32 = jnp.float32


def _leaky(v):
    return jnp.where(v >= 0.0, v, 0.2 * v)


def _dense1_body(x_ref, w_ref, asw_ref, adw_ref,
                 h0_ref, h1_ref, as_ref, ad_ref, sum_s, sq_s):
    p = pl.program_id(0)
    i = pl.program_id(1)

    @pl.when(p == 0)
    def _():
        xb = x_ref[...]
        s = jnp.sum(xb, axis=0, keepdims=True)
        q = jnp.sum(xb * xb, axis=0, keepdims=True)

        @pl.when(i == 0)
        def _():
            sum_s[...] = s
            sq_s[...] = q

        @pl.when(i != 0)
        def _():
            sum_s[...] = sum_s[...] + s
            sq_s[...] = sq_s[...] + q

    @pl.when(p == 1)
    def _():
        mu = sum_s[...] * (1.0 / N)
        var = sq_s[...] * (1.0 / N) - mu * mu
        inv = lax.rsqrt(var + 1e-5)
        xn = (x_ref[...] - mu) * inv
        h = jnp.dot(xn, w_ref[...], preferred_element_type=_f32)
        ones = jnp.ones((h.shape[0], DHP - DH), _f32)
        h0_ref[...] = jnp.concatenate([h[:, :DH], ones], axis=1)
        h1_ref[...] = jnp.concatenate([h[:, DH:], ones], axis=1)
        as_ref[...] = jnp.dot(h, asw_ref[...], preferred_element_type=_f32)
        ad_ref[...] = jnp.dot(h, adw_ref[...], preferred_element_type=_f32)


def _dense2_body(c0_ref, c1_ref, as1_ref, ad1_ref, h0_ref, h1_ref,
                 bias_ref, w_ref, asw_ref, adw_ref,
                 g0_ref, g1_ref, as2_ref, ad2_ref):
    wself = jnp.exp(_leaky(as1_ref[...] + ad1_ref[...]))
    den = c0_ref[:, DH:DH + 1] + wself + 1e-16
    hb = jnp.concatenate([h0_ref[:, :DH], h1_ref[:, :DH]], axis=1)
    cb = jnp.concatenate([c0_ref[:, :DH], c1_ref[:, :DH]], axis=1)
    out1 = (cb + wself * hb) / den + bias_ref[0:1, :]
    h2 = jnp.dot(out1, w_ref[...], preferred_element_type=_f32)
    ones = jnp.ones((h2.shape[0], DHP - DH), _f32)
    g0_ref[...] = jnp.concatenate([h2[:, :DH], ones], axis=1)
    g1_ref[...] = jnp.concatenate([h2[:, DH:], ones], axis=1)
    as2_ref[...] = jnp.dot(h2, asw_ref[...], preferred_element_type=_f32)
    ad2_ref[...] = jnp.dot(h2, adw_ref[...], preferred_element_type=_f32)


def _final_body(c0_ref, c1_ref, as2_ref, ad2_ref, h0_ref, h1_ref,
                bias_ref, out_ref):
    wself = jnp.exp(_leaky(as2_ref[...] + ad2_ref[...]))
    den = c0_ref[:, DH:DH + 1] + wself + 1e-16
    hb = jnp.concatenate([h0_ref[:, :DH], h1_ref[:, :DH]], axis=1)
    cb = jnp.concatenate([c0_ref[:, :DH], c1_ref[:, :DH]], axis=1)
    out_ref[...] = (cb + wself * hb) / den + bias_ref[0:1, :]


def _row_spec(width=D):
    return pl.BlockSpec((RB, width), lambda i: (i, 0))


def _full_spec(shape):
    return pl.BlockSpec(shape, lambda i: tuple(0 for _ in shape))


def _row_spec2(width):
    return pl.BlockSpec((RB, width), lambda p, i: (i, 0))


_dense1_call = pl.pallas_call(
    _dense1_body,
    grid=(2, GRID),
    in_specs=[_row_spec2(D),
              pl.BlockSpec((D, D), lambda p, i: (0, 0)),
              pl.BlockSpec((D, 1), lambda p, i: (0, 0)),
              pl.BlockSpec((D, 1), lambda p, i: (0, 0))],
    out_specs=[_row_spec2(DHP), _row_spec2(DHP), _row_spec2(1),
               _row_spec2(1)],
    out_shape=[jax.ShapeDtypeStruct((N, DHP), _f32),
               jax.ShapeDtypeStruct((N, DHP), _f32),
               jax.ShapeDtypeStruct((N, 1), _f32),
               jax.ShapeDtypeStruct((N, 1), _f32)],
    scratch_shapes=[pltpu.VMEM((1, D), _f32), pltpu.VMEM((1, D), _f32)],
)

_dense2_call = pl.pallas_call(
    _dense2_body,
    grid=(GRID,),
    in_specs=[_row_spec(DHP), _row_spec(DHP), _row_spec(1), _row_spec(1),
              _row_spec(DHP), _row_spec(DHP),
              _full_spec((1, D)), _full_spec((D, D)),
              _full_spec((D, 1)), _full_spec((D, 1))],
    out_specs=[_row_spec(DHP), _row_spec(DHP), _row_spec(1), _row_spec(1)],
    out_shape=[jax.ShapeDtypeStruct((N, DHP), _f32),
               jax.ShapeDtypeStruct((N, DHP), _f32),
               jax.ShapeDtypeStruct((N, 1), _f32),
               jax.ShapeDtypeStruct((N, 1), _f32)],
)

_final_call = pl.pallas_call(
    _final_body,
    grid=(GRID,),
    in_specs=[_row_spec(DHP), _row_spec(DHP), _row_spec(1), _row_spec(1),
              _row_spec(DHP), _row_spec(DHP), _full_spec((1, D))],
    out_specs=_row_spec(),
    out_shape=jax.ShapeDtypeStruct((N, D), _f32),
)


def _part_body(src_hbm, dst_hbm, addr_hbm,
               src0_hbm, dst0_hbm, cnt0_hbm, src1_hbm, dst1_hbm, cnt1_hbm,
               src_v, dst_v, addr_v, src0_v, dst0_v, src1_v, dst1_v, cnt_v):
    c = lax.axis_index("c")
    t = lax.axis_index("s")

    @pl.when(c == 0)
    def _():
        pltpu.sync_copy(src_hbm.at[t], src_v)
        pltpu.sync_copy(dst_hbm.at[t], dst_v)
        pltpu.sync_copy(addr_hbm.at[t], addr_v)

        def cstep(i, pos):
            pos0, pos1 = pos
            off = i * 16
            sv = src_v[pl.ds(off, 16)]
            dv = dst_v[pl.ds(off, 16)]
            av = addr_v[pl.ds(off, 16)]
            m0 = av == 0
            m1 = av == 1
            plsc.store_compressed(src0_v.at[pl.ds(pos0, 16)], sv, mask=m0)
            plsc.store_compressed(dst0_v.at[pl.ds(pos0, 16)], dv, mask=m0)
            plsc.store_compressed(src1_v.at[pl.ds(pos1, 16)], sv, mask=m1)
            plsc.store_compressed(dst1_v.at[pl.ds(pos1, 16)], dv, mask=m1)
            n0 = plsc.all_reduce_population_count(m0)[0]
            n1 = plsc.all_reduce_population_count(m1)[0]
            return pos0 + n0, pos1 + n1

        pos0, pos1 = lax.fori_loop(0, CHUNK // 16, cstep, (0, 0))

        sent = jnp.full((16,), MASKBIT, jnp.int32)
        zero = jnp.zeros((16,), jnp.int32)
        for k in range(SUP // 16):
            src0_v[pl.ds(pos0 + k * 16, 16)] = sent
            dst0_v[pl.ds(pos0 + k * 16, 16)] = zero
            src1_v[pl.ds(pos1 + k * 16, 16)] = sent
            dst1_v[pl.ds(pos1 + k * 16, 16)] = zero

        pltpu.sync_copy(src0_v, src0_hbm.at[t])
        pltpu.sync_copy(dst0_v, dst0_hbm.at[t])
        pltpu.sync_copy(src1_v, src1_hbm.at[t])
        pltpu.sync_copy(dst1_v, dst1_hbm.at[t])
        cnt_v[...] = jnp.full((16,), (pos0 + SUP - 1) // SUP, jnp.int32)
        pltpu.sync_copy(cnt_v, cnt0_hbm.at[t])
        cnt_v[...] = jnp.full((16,), (pos1 + SUP - 1) // SUP, jnp.int32)
        pltpu.sync_copy(cnt_v, cnt1_hbm.at[t])


_part_call = pl.kernel(
    _part_body,
    out_type=[jax.ShapeDtypeStruct((16, LCAP), jnp.int32),
              jax.ShapeDtypeStruct((16, LCAP), jnp.int32),
              jax.ShapeDtypeStruct((16, 16), jnp.int32),
              jax.ShapeDtypeStruct((16, LCAP), jnp.int32),
              jax.ShapeDtypeStruct((16, LCAP), jnp.int32),
              jax.ShapeDtypeStruct((16, 16), jnp.int32)],
    mesh=plsc.VectorSubcoreMesh(core_axis_name="c", subcore_axis_name="s"),
    compiler_params=pltpu.CompilerParams(needs_layout_passes=False,
                                         use_tc_tiling_on_sc=False),
    scratch_types=[
        pltpu.VMEM((CHUNK,), jnp.int32),
        pltpu.VMEM((CHUNK,), jnp.int32),
        pltpu.VMEM((CHUNK,), jnp.int32),
        pltpu.VMEM((LCAP,), jnp.int32),
        pltpu.VMEM((LCAP,), jnp.int32),
        pltpu.VMEM((LCAP,), jnp.int32),
        pltpu.VMEM((LCAP,), jnp.int32),
        pltpu.VMEM((16,), jnp.int32),
    ],
)


def _wc_body(psd_hbm, cnt_hbm, asrc_hbm, adst_hbm,
             wl_hbm, psdc_hbm,
             psd_s, w_s, asrc_v, adst_v, cnt_v):
    c = lax.axis_index("c")
    t = lax.axis_index("s")

    @pl.when(c == 0)
    def _():
        pltpu.sync_copy(asrc_hbm, asrc_v)
        pltpu.sync_copy(adst_hbm, adst_v)
        pltpu.sync_copy(cnt_hbm.at[t], cnt_v)
        nsup = cnt_v[...][0]

        def sup_step(sup, _):
            pltpu.sync_copy(psd_hbm.at[t, sup], psd_s)

            def wstep(i, _):
                r = i // (PCH // 16)
                o = pl.ds((i % (PCH // 16)) * 16, 16)
                pv = psd_s[0, r, o]
                srcv = jnp.bitwise_and(pv, MASKBIT - 1)
                maskv = pv < MASKBIT
                av = plsc.load_gather(asrc_v, [srcv])
                bv = plsc.load_gather(adst_v, [psd_s[1, r, o]])
                e = av + bv
                e = jnp.where(e >= 0.0, e, 0.2 * e)
                w_s[pl.ds(i * 16, 16)] = jnp.where(maskv, jnp.exp(e), 0.0)
                psd_s[0, r, o] = srcv
                return 0

            lax.fori_loop(0, SUP // 16, wstep, 0)
            pltpu.sync_copy(w_s, wl_hbm.at[t, sup])
            pltpu.sync_copy(psd_s, psdc_hbm.at[t, sup])
            return 0

        lax.fori_loop(0, nsup, sup_step, 0)


_wc_call = pl.kernel(
    _wc_body,
    out_type=[jax.ShapeDtypeStruct((16, NSUPC, SUP), _f32),
              jax.ShapeDtypeStruct((16, NSUPC, 2, SCH, PCH), jnp.int32)],
    mesh=plsc.VectorSubcoreMesh(core_axis_name="c", subcore_axis_name="s"),
    compiler_params=pltpu.CompilerParams(needs_layout_passes=False,
                                         use_tc_tiling_on_sc=False),
    scratch_types=[
        pltpu.VMEM((2, SCH, PCH), jnp.int32),
        pltpu.VMEM((SUP,), _f32),
        pltpu.VMEM((N,), _f32),
        pltpu.VMEM((N,), _f32),
        pltpu.VMEM((16,), jnp.int32),
    ],
)


def _sc_body(psd_hbm, wl_hbm, cnt_hbm, h0_hbm, h1_hbm,
             c0_hbm, c1_hbm,
             psd_s, w_s, rows_a, rows_b, cnt_v,
             sem_ga, sem_gb, sem_sa, sem_sb, acc_sh):
    c = lax.axis_index("c")
    t = lax.axis_index("s")

    pltpu.sync_copy(cnt_hbm.at[t], cnt_v)
    nsup = cnt_v[...][0]

    def z2(r, _):
        for k in range(DHP // 16):
            rows_a[r, pl.ds(k * 16, 16)] = jnp.zeros((16,), _f32)
        return 0

    lax.fori_loop(0, PCH, z2, 0)
    part = 0
    while part < SLICE:
        sz = min(PCH, SLICE - part)
        pltpu.sync_copy(rows_a.at[pl.ds(0, sz)],
                        acc_sh.at[pl.ds(t * SLICE + part, sz)])
        part += sz
    plsc.subcore_barrier()

    rows = (rows_a, rows_b)
    gsem = (sem_ga, sem_gb)
    ssem = (sem_sa, sem_sb)

    def _gather(j, buf, sem):
        idx = psd_s.at[0, j]
        d0 = pltpu.make_async_copy(h0_hbm.at[idx], buf, sem)
        d1 = pltpu.make_async_copy(h1_hbm.at[idx], buf, sem)

        @pl.when(c == 0)
        def _():
            d0.start()

        @pl.when(c == 1)
        def _():
            d1.start()

        return d0

    def sup_step(sup, _):
        pltpu.sync_copy(psd_hbm.at[t, sup], psd_s)
        pltpu.sync_copy(wl_hbm.at[t, sup], w_s)

        gw = [None, None]
        sw = [None, None]
        gw[0] = _gather(0, rows[0], gsem[0])
        for j in range(SCH):
            b = j & 1
            gw[b].wait()
            if j + 1 < SCH:
                if sw[1 - b] is not None:
                    sw[1 - b].wait()
                gw[1 - b] = _gather(j + 1, rows[1 - b], gsem[1 - b])

            def sstep(i, _, _b=b, _j=j):
                wv = w_s[pl.ds(_j * PCH + i * 16, 16)]
                for jj in range(16):
                    wj = wv[jj]
                    r = i * 16 + jj
                    for k in range(DHP // 16):
                        o = pl.ds(k * 16, 16)
                        rows[_b][r, o] = rows[_b][r, o] * wj
                return 0

            lax.fori_loop(0, PCH // 16, sstep, 0)
            sw[b] = pltpu.async_copy(rows[b], acc_sh.at[psd_s.at[1, j]],
                                     ssem[b], add=True)
        sw[0].wait()
        sw[1].wait()
        return 0

    lax.fori_loop(0, nsup, sup_step, 0)
    plsc.subcore_barrier()

    wsl = pl.ds(t * SLICE, SLICE)

    @pl.when(c == 0)
    def _():
        pltpu.sync_copy(acc_sh.at[wsl], c0_hbm.at[wsl])

    @pl.when(c == 1)
    def _():
        pltpu.sync_copy(acc_sh.at[wsl], c1_hbm.at[wsl])


_sc_call = pl.kernel(
    _sc_body,
    out_type=[jax.ShapeDtypeStruct((NACC, DHP), _f32),
              jax.ShapeDtypeStruct((NACC, DHP), _f32)],
    mesh=plsc.VectorSubcoreMesh(core_axis_name="c", subcore_axis_name="s"),
    compiler_params=pltpu.CompilerParams(needs_layout_passes=False,
                                         use_tc_tiling_on_sc=False),
    scratch_types=[
        pltpu.VMEM((2, SCH, PCH), jnp.int32),
        pltpu.VMEM((SUP,), _f32),
        pltpu.VMEM((PCH, DHP), _f32),
        pltpu.VMEM((PCH, DHP), _f32),
        pltpu.VMEM((16,), jnp.int32),
        pltpu.SemaphoreType.DMA,
        pltpu.SemaphoreType.DMA,
        pltpu.SemaphoreType.DMA,
        pltpu.SemaphoreType.DMA,
        pltpu.VMEM_SHARED((NACC, DHP), _f32),
    ],
)


def kernel(x, edge_index, edge_addr, W_inc, att_src_inc, att_dst_inc,
           bias_inc, W_near, att_src_near, att_dst_near, bias_near):
    src = edge_index[0]
    dst = edge_index[1]
    pad = EPAD - E
    srcp = jnp.concatenate([src, jnp.zeros((pad,), jnp.int32)])
    dstp = jnp.concatenate([dst, jnp.zeros((pad,), jnp.int32)])
    addrp = jnp.concatenate([edge_addr, jnp.full((pad,), 2, jnp.int32)])

    src0, dst0, cnt0, src1, dst1, cnt1 = _part_call(
        srcp.reshape(16, CHUNK), dstp.reshape(16, CHUNK),
        addrp.reshape(16, CHUNK))
    psd0 = jnp.stack([src0.reshape(16, NSUPC, SCH, PCH),
                      dst0.reshape(16, NSUPC, SCH, PCH)], axis=2)
    psd1 = jnp.stack([src1.reshape(16, NSUPC, SCH, PCH),
                      dst1.reshape(16, NSUPC, SCH, PCH)], axis=2)

    h0, h1, a_s1, a_d1 = _dense1_call(
        x, W_inc, att_src_inc.reshape(D, 1), att_dst_inc.reshape(D, 1))

    wl0, psdc0 = _wc_call(psd0, cnt0, a_s1.reshape(N), a_d1.reshape(N))
    c0, c1 = _sc_call(psdc0, wl0, cnt0, h0, h1)

    g0, g1, a_s2, a_d2 = _dense2_call(
        c0[:N], c1[:N], a_s1, a_d1, h0, h1,
        bias_inc.reshape(1, D), W_near, att_src_near.reshape(D, 1),
        att_dst_near.reshape(D, 1))

    wl1, psdc1 = _wc_call(psd1, cnt1, a_s2.reshape(N), a_d2.reshape(N))
    e0, e1 = _sc_call(psdc1, wl1, cnt1, g0, g1)

    out = _final_call(e0[:N], e1[:N], a_s2, a_d2,
                      g0, g1, bias_near.reshape(1, D))
    return out
